# baseline (speedup 1.0000x reference)
import jax
import jax.numpy as jnp
from jax.experimental import pallas as pl
from jax.experimental.pallas import tpu as pltpu

N_HE = 20
K_TOP = 5
BLK = 4000


def _pass1_kernel(x_ref, w1_ref, b1_ref, c_ref, p_ref, hs_ref, s2_ref, de_ref,
                  hsc_ref, psc_ref):
    i = pl.program_id(0)
    nlast = pl.num_programs(0) - 1
    xb = x_ref[...]
    w = w1_ref[...]
    b = b1_ref[...]
    p = jnp.tanh(
        jax.lax.dot_general(xb, w, (((1,), (1,)), ((), ())),
                            preferred_element_type=jnp.float32) + b)

    c = c_ref[...]
    cn = jnp.sqrt(jnp.sum(c * c, axis=1, keepdims=True))
    cn = jnp.where(cn > 0, cn, 1.0)
    cnorm = c / cn

    pp = p * p
    ones = jnp.ones((8, p.shape[1]), jnp.float32)
    pn2 = jax.lax.dot_general(ones, pp, (((1,), (1,)), ((), ())),
                              preferred_element_type=jnp.float32)[:1]
    pn = jnp.sqrt(pn2)
    pninv = jnp.where(pn > 0, 1.0 / pn, 1.0)

    st = jax.lax.dot_general(cnorm, p, (((1,), (1,)), ((), ())),
                             preferred_element_type=jnp.float32) * pninv
    st = jnp.clip(st, -1.0, 1.0)

    idx = jax.lax.broadcasted_iota(jnp.int32, st.shape, 0)
    bits = jax.lax.bitcast_convert_type(st, jnp.int32)
    mono = bits ^ (jax.lax.shift_right_arithmetic(bits, 31)
                   & jnp.int32(0x7FFFFFFF))
    keyi = (((mono + jnp.int32(1065353217)) & jnp.int32(~31))
            | (jnp.int32(31) - idx))
    key = jax.lax.bitcast_convert_type(keyi, jnp.float32)
    kmsel = jnp.zeros(st.shape, jnp.bool_)
    for _ in range(K_TOP):
        mk = jnp.max(key, axis=0, keepdims=True)
        first = key == mk
        kmsel = kmsel | first
        key = jnp.where(first, -1.0, key)

    ht = jnp.where(kmsel, st, 0.0)
    dv = jnp.sum(ht, axis=0, keepdims=True)
    dvis = 1.0 / (jnp.sqrt(dv) + 1e-6)
    hst = ht * dvis

    p_ref[...] = p.astype(jnp.bfloat16)
    hs_ref[0] = hst.astype(jnp.bfloat16)

    de = jnp.sum(ht, axis=1, keepdims=True)

    @pl.when(i == 0)
    def _():
        de_ref[...] = de
        s2_ref[...] = jnp.zeros_like(s2_ref)

    @pl.when(i > 0)
    def _():
        de_ref[...] = de_ref[...] + de
        s2_ref[...] = s2_ref[...] + jax.lax.dot_general(
            hsc_ref[...], psc_ref[...], (((1,), (0,)), ((), ())),
            preferred_element_type=jnp.float32)

    hsc_ref[...] = hst
    psc_ref[...] = p

    @pl.when(i == nlast)
    def _():
        s2_ref[...] = s2_ref[...] + jax.lax.dot_general(
            hst, p, (((1,), (0,)), ((), ())),
            preferred_element_type=jnp.float32)


def _pass2_kernel(p_ref, hs_ref, s2_ref, de_ref, a_ref, o_ref):
    p = p_ref[...].astype(jnp.float32)
    hst = hs_ref[0].astype(jnp.float32)
    s2 = s2_ref[...]
    de = de_ref[...]
    a = a_ref[...]
    step3 = s2 * (1.0 / (de + 1e-6))
    step = jax.lax.dot_general(hst, step3, (((0,), (0,)), ((), ())),
                               preferred_element_type=jnp.float32)
    o_ref[...] = (2.0 - a) * p + a * step


def kernel(x, W1, b1, centers, alpha):
    x = x.astype(jnp.float32)
    n, d_in = x.shape
    d_out = W1.shape[0]
    grid = (n // BLK,)

    p, hs, s2, de = pl.pallas_call(
        _pass1_kernel,
        grid=grid,
        in_specs=[
            pl.BlockSpec((BLK, d_in), lambda i: (i, 0)),
            pl.BlockSpec((d_out, d_in), lambda i: (0, 0)),
            pl.BlockSpec((1, d_out), lambda i: (0, 0)),
            pl.BlockSpec((N_HE, d_out), lambda i: (0, 0)),
        ],
        out_specs=[
            pl.BlockSpec((BLK, d_out), lambda i: (i, 0)),
            pl.BlockSpec((1, N_HE, BLK), lambda i: (i, 0, 0)),
            pl.BlockSpec((N_HE, d_out), lambda i: (0, 0)),
            pl.BlockSpec((N_HE, 1), lambda i: (0, 0)),
        ],
        out_shape=[
            jax.ShapeDtypeStruct((n, d_out), jnp.bfloat16),
            jax.ShapeDtypeStruct((n // BLK, N_HE, BLK), jnp.bfloat16),
            jax.ShapeDtypeStruct((N_HE, d_out), jnp.float32),
            jax.ShapeDtypeStruct((N_HE, 1), jnp.float32),
        ],
        scratch_shapes=[
            pltpu.VMEM((N_HE, BLK), jnp.float32),
            pltpu.VMEM((BLK, d_out), jnp.float32),
        ],
        compiler_params=pltpu.CompilerParams(
            dimension_semantics=("arbitrary",)),
    )(x, W1, b1.reshape(1, d_out), centers)

    out = pl.pallas_call(
        _pass2_kernel,
        grid=grid,
        in_specs=[
            pl.BlockSpec((BLK, d_out), lambda i: (i, 0)),
            pl.BlockSpec((1, N_HE, BLK), lambda i: (i, 0, 0)),
            pl.BlockSpec((N_HE, d_out), lambda i: (0, 0)),
            pl.BlockSpec((N_HE, 1), lambda i: (0, 0)),
            pl.BlockSpec((1, 1), lambda i: (0, 0)),
        ],
        out_specs=pl.BlockSpec((BLK, d_out), lambda i: (i, 0)),
        out_shape=jax.ShapeDtypeStruct((n, d_out), jnp.float32),
        compiler_params=pltpu.CompilerParams(
            dimension_semantics=("parallel",)),
    )(p, hs, s2, de, jnp.asarray(alpha, jnp.float32).reshape(1, 1))

    return out


# speedup vs baseline: 1.0330x; 1.0330x over previous
import jax
import jax.numpy as jnp
from jax.experimental import pallas as pl
from jax.experimental.pallas import tpu as pltpu

N_HE = 20
K_TOP = 5
BLK = 4000


def _pass1_kernel(x_ref, w1_ref, b1_ref, c_ref, p_ref, hs_ref, s2_ref, de_ref):
    i = pl.program_id(0)
    xb = x_ref[...]
    w = w1_ref[...]
    b = b1_ref[...]
    p = jnp.tanh(
        jax.lax.dot_general(xb, w, (((1,), (1,)), ((), ())),
                            preferred_element_type=jnp.float32) + b)

    c = c_ref[...]
    cn = jnp.sqrt(jnp.sum(c * c, axis=1, keepdims=True))
    cn = jnp.where(cn > 0, cn, 1.0)
    cnorm = c / cn

    pp = p * p
    ones = jnp.ones((8, p.shape[1]), jnp.float32)
    pn2 = jax.lax.dot_general(ones, pp, (((1,), (1,)), ((), ())),
                              preferred_element_type=jnp.float32)[:1]
    pn = jnp.sqrt(pn2)
    pninv = jnp.where(pn > 0, 1.0 / pn, 1.0)

    st = jax.lax.dot_general(cnorm, p, (((1,), (1,)), ((), ())),
                             preferred_element_type=jnp.float32) * pninv
    st = jnp.clip(st, -1.0, 1.0)

    idx = jax.lax.broadcasted_iota(jnp.int32, st.shape, 0)
    bits = jax.lax.bitcast_convert_type(st, jnp.int32)
    mono = bits ^ (jax.lax.shift_right_arithmetic(bits, 31)
                   & jnp.int32(0x7FFFFFFF))
    keyi = (((mono + jnp.int32(1065353217)) & jnp.int32(~31))
            | (jnp.int32(31) - idx))
    key = jax.lax.bitcast_convert_type(keyi, jnp.float32)
    kmsel = jnp.zeros(st.shape, jnp.bool_)
    for _ in range(K_TOP):
        mk = jnp.max(key, axis=0, keepdims=True)
        first = key == mk
        kmsel = kmsel | first
        key = jnp.where(first, -1.0, key)

    ht = jnp.where(kmsel, st, 0.0)
    dv = jnp.sum(ht, axis=0, keepdims=True)
    dvis = 1.0 / (jnp.sqrt(dv) + 1e-6)
    hst = ht * dvis

    p_ref[...] = p.astype(jnp.bfloat16)
    hs_ref[0] = hst.astype(jnp.bfloat16)

    de = jnp.sum(ht, axis=1, keepdims=True)
    s2 = jax.lax.dot_general(hst, p, (((1,), (0,)), ((), ())),
                             preferred_element_type=jnp.float32)

    @pl.when(i == 0)
    def _():
        de_ref[...] = de
        s2_ref[...] = s2

    @pl.when(i > 0)
    def _():
        de_ref[...] = de_ref[...] + de
        s2_ref[...] = s2_ref[...] + s2


def _pass2_kernel(p_ref, hs_ref, s2_ref, de_ref, a_ref, o_ref):
    p = p_ref[...].astype(jnp.float32)
    hst = hs_ref[0].astype(jnp.float32)
    s2 = s2_ref[...]
    de = de_ref[...]
    a = a_ref[...]
    step3 = s2 * (1.0 / (de + 1e-6))
    step = jax.lax.dot_general(hst, step3, (((0,), (0,)), ((), ())),
                               preferred_element_type=jnp.float32)
    o_ref[...] = (2.0 - a) * p + a * step


def kernel(x, W1, b1, centers, alpha):
    x = x.astype(jnp.float32)
    n, d_in = x.shape
    d_out = W1.shape[0]
    grid = (n // BLK,)

    p, hs, s2, de = pl.pallas_call(
        _pass1_kernel,
        grid=grid,
        in_specs=[
            pl.BlockSpec((BLK, d_in), lambda i: (i, 0)),
            pl.BlockSpec((d_out, d_in), lambda i: (0, 0)),
            pl.BlockSpec((1, d_out), lambda i: (0, 0)),
            pl.BlockSpec((N_HE, d_out), lambda i: (0, 0)),
        ],
        out_specs=[
            pl.BlockSpec((BLK, d_out), lambda i: (i, 0)),
            pl.BlockSpec((1, N_HE, BLK), lambda i: (i, 0, 0)),
            pl.BlockSpec((N_HE, d_out), lambda i: (0, 0)),
            pl.BlockSpec((N_HE, 1), lambda i: (0, 0)),
        ],
        out_shape=[
            jax.ShapeDtypeStruct((n, d_out), jnp.bfloat16),
            jax.ShapeDtypeStruct((n // BLK, N_HE, BLK), jnp.bfloat16),
            jax.ShapeDtypeStruct((N_HE, d_out), jnp.float32),
            jax.ShapeDtypeStruct((N_HE, 1), jnp.float32),
        ],
        compiler_params=pltpu.CompilerParams(
            dimension_semantics=("arbitrary",)),
    )(x, W1, b1.reshape(1, d_out), centers)

    out = pl.pallas_call(
        _pass2_kernel,
        grid=grid,
        in_specs=[
            pl.BlockSpec((BLK, d_out), lambda i: (i, 0)),
            pl.BlockSpec((1, N_HE, BLK), lambda i: (i, 0, 0)),
            pl.BlockSpec((N_HE, d_out), lambda i: (0, 0)),
            pl.BlockSpec((N_HE, 1), lambda i: (0, 0)),
            pl.BlockSpec((1, 1), lambda i: (0, 0)),
        ],
        out_specs=pl.BlockSpec((BLK, d_out), lambda i: (i, 0)),
        out_shape=jax.ShapeDtypeStruct((n, d_out), jnp.float32),
        compiler_params=pltpu.CompilerParams(
            dimension_semantics=("parallel",)),
    )(p, hs, s2, de, jnp.asarray(alpha, jnp.float32).reshape(1, 1))

    return out
